# Initial kernel scaffold; baseline (speedup 1.0000x reference)
import jax
import jax.numpy as jnp
from jax.experimental import pallas as pl


def kernel(idx, wte, wpe):
    raise NotImplementedError("write your pallas kernel here")


# speedup vs baseline: 1.2012x; 1.2012x over previous
import functools

import jax
import jax.numpy as jnp
from jax import lax
from jax.experimental import pallas as pl
from jax.experimental.pallas import tpu as pltpu
from jax.experimental.pallas import tpu_sc as plsc

B, T, C = 4, 2048, 768
N = B * T
NC, NS = 2, 16
NW = NC * NS
PER_W = N // NW
R = 32
NCH = PER_W // R
LPR = C // 16

_mesh = plsc.VectorSubcoreMesh(
    core_axis_name="c", subcore_axis_name="s", num_cores=NC, num_subcores=NS
)


@functools.partial(
    pl.kernel,
    out_type=jax.ShapeDtypeStruct((N, C), jnp.float32),
    mesh=_mesh,
    scratch_types=[
        pltpu.VMEM((NCH, R), jnp.int32),
        pltpu.VMEM((R, C), jnp.float32),
        pltpu.VMEM((R, C), jnp.float32),
        pltpu.VMEM((R, C), jnp.float32),
        pltpu.VMEM((R, C), jnp.float32),
        pltpu.SemaphoreType.DMA,
        pltpu.SemaphoreType.DMA,
        pltpu.SemaphoreType.DMA,
        pltpu.SemaphoreType.DMA,
        pltpu.SemaphoreType.DMA,
        pltpu.SemaphoreType.DMA,
    ],
)
def _encode(idx_hbm, wte_hbm, wpe_hbm, out_hbm,
            idx_v, g0, g1, p0, p1, gs0, gs1, ws0, ws1, os0, os1):
    wid = lax.axis_index("s") * NC + lax.axis_index("c")
    base = wid * PER_W
    t0 = lax.rem(base, T)
    pltpu.sync_copy(idx_hbm.at[pl.ds(wid * NCH, NCH)], idx_v)

    gbufs, pbufs = (g0, g1), (p0, p1)
    gsems, wsems, osems = (gs0, gs1), (ws0, ws1), (os0, os1)
    g_h = [None, None]
    w_h = [None, None]
    o_h = [None, None]

    def start(ch):
        b = ch & 1
        g_h[b] = pltpu.async_copy(wte_hbm.at[idx_v.at[ch]], gbufs[b], gsems[b])
        w_h[b] = pltpu.async_copy(
            wpe_hbm.at[pl.ds(t0 + ch * R, R)], pbufs[b], wsems[b]
        )

    start(0)
    for ch in range(NCH):
        b = ch & 1
        nb = b ^ 1
        if ch + 1 < NCH:
            if o_h[nb] is not None:
                o_h[nb].wait()
            start(ch + 1)
        g_h[b].wait()
        w_h[b].wait()
        gbuf, pbuf = gbufs[b], pbufs[b]

        def add_row(r, _):
            for j in range(LPR):
                sl = pl.ds(j * 16, 16)
                plsc.addupdate(gbuf.at[r, sl], pbuf[r, sl])
            return _

        lax.fori_loop(0, R, add_row, None)
        o_h[b] = pltpu.async_copy(
            gbufs[b], out_hbm.at[pl.ds(base + ch * R, R)], osems[b]
        )
    for h in o_h:
        if h is not None:
            h.wait()


def kernel(idx, wte, wpe):
    idx_flat = idx.reshape(N // R, R).astype(jnp.int32)
    out = _encode(idx_flat, wte, wpe)
    return out.reshape(B, T, C)


# speedup vs baseline: 1.3018x; 1.0837x over previous
import functools

import jax
import jax.numpy as jnp
from jax import lax
from jax.experimental import pallas as pl
from jax.experimental.pallas import tpu as pltpu
from jax.experimental.pallas import tpu_sc as plsc

B, T, C = 4, 2048, 768
N = B * T
NC, NS = 2, 16
NW = NC * NS
R = 32
TPW = T // NW
NTC = TPW // R
NCH = NTC * B
LPR = C // 16

_mesh = plsc.VectorSubcoreMesh(
    core_axis_name="c", subcore_axis_name="s", num_cores=NC, num_subcores=NS
)


@functools.partial(
    pl.kernel,
    out_type=jax.ShapeDtypeStruct((N, C), jnp.float32),
    mesh=_mesh,
    scratch_types=[
        pltpu.VMEM((NCH, R), jnp.int32),
        pltpu.VMEM((R, C), jnp.float32),
        pltpu.VMEM((R, C), jnp.float32),
        pltpu.VMEM((R, C), jnp.float32),
        pltpu.VMEM((R, C), jnp.float32),
        pltpu.SemaphoreType.DMA,
        pltpu.SemaphoreType.DMA,
        pltpu.SemaphoreType.DMA,
        pltpu.SemaphoreType.DMA,
        pltpu.SemaphoreType.DMA,
        pltpu.SemaphoreType.DMA,
    ],
)
def _encode(idx_hbm, wte_hbm, wpe_hbm, out_hbm,
            idx_v, g0, g1, p0, p1, gs0, gs1, ws0, ws1, os0, os1):
    wid = lax.axis_index("s") * NC + lax.axis_index("c")
    t0 = wid * TPW
    pltpu.sync_copy(idx_hbm.at[pl.ds(wid * NCH, NCH)], idx_v)

    gbufs, pbufs = (g0, g1), (p0, p1)
    gsems, wsems, osems = (gs0, gs1), (ws0, ws1), (os0, os1)
    w_h = [
        pltpu.async_copy(wpe_hbm.at[pl.ds(t0 + tc * R, R)], pbufs[tc], wsems[tc])
        for tc in range(NTC)
    ]
    g_h = [None, None]
    o_h = [None, None]

    def start(ch):
        b = ch & 1
        g_h[b] = pltpu.async_copy(wte_hbm.at[idx_v.at[ch]], gbufs[b], gsems[b])

    start(0)
    for ch in range(NCH):
        tc, batch = ch // B, ch % B
        b = ch & 1
        nb = b ^ 1
        if ch + 1 < NCH:
            if o_h[nb] is not None:
                o_h[nb].wait()
            start(ch + 1)
        g_h[b].wait()
        if w_h[tc] is not None:
            w_h[tc].wait()
            w_h[tc] = None
        gbuf, pbuf = gbufs[b], pbufs[tc]

        def add_row(r, _):
            for j in range(LPR):
                sl = pl.ds(j * 16, 16)
                plsc.addupdate(gbuf.at[r, sl], pbuf[r, sl])
            return _

        lax.fori_loop(0, R, add_row, None)
        o_h[b] = pltpu.async_copy(
            gbufs[b], out_hbm.at[pl.ds(batch * T + t0 + tc * R, R)], osems[b]
        )
    for h in o_h:
        if h is not None:
            h.wait()


def kernel(idx, wte, wpe):
    idx_r = (
        idx.astype(jnp.int32)
        .reshape(B, NW, NTC, R)
        .transpose(1, 2, 0, 3)
        .reshape(N // R, R)
    )
    out = _encode(idx_r, wte, wpe)
    return out.reshape(B, T, C)
